# baseline (speedup 1.0000x reference)
import jax
import jax.numpy as jnp
from jax import lax
from jax.experimental import pallas as pl
from jax.experimental.pallas import tpu as pltpu
from jax.experimental.pallas import tpu_sc as plsc

N = 10000
E = 160000
D = 256
H = 128
NEG = 0.2

L = 16
NC = 2
NS = 16
NW = NC * NS

N_PAD = 10240
E_TOT = E + N
E_PAD = 172032
K1 = 64
EP1 = E_PAD // NW
NB1 = EP1 // K1
K2 = 48
EP2 = E_PAD // NS
NB2 = EP2 // K2
R = 1024
DR = N_PAD // NS
DCH = 32


def _proj_body(x_ref, wl_ref, bl_ref, wr_ref, br_ref,
               ol_ref, olb_ref, orb_ref):
    xb = x_ref[...]
    xl = jnp.dot(xb, wl_ref[...],
                 preferred_element_type=jnp.float32) + bl_ref[...]
    xr = jnp.dot(xb, wr_ref[...],
                 preferred_element_type=jnp.float32) + br_ref[...]
    ol_ref[...] = xl
    olb_ref[...] = xl.astype(jnp.bfloat16)
    orb_ref[...] = xr.astype(jnp.bfloat16)


_proj = pl.pallas_call(
    _proj_body,
    grid=(2, N_PAD // R),
    in_specs=[
        pl.BlockSpec((R, D), lambda h, i: (i, 0)),
        pl.BlockSpec((D, H), lambda h, i: (0, h)),
        pl.BlockSpec((1, H), lambda h, i: (0, h)),
        pl.BlockSpec((D, H), lambda h, i: (0, h)),
        pl.BlockSpec((1, H), lambda h, i: (0, h)),
    ],
    out_specs=[
        pl.BlockSpec((R, H), lambda h, i: (h * (N_PAD // R) + i, 0)),
        pl.BlockSpec((R, H), lambda h, i: (i, h)),
        pl.BlockSpec((R, H), lambda h, i: (i, h)),
    ],
    out_shape=[jax.ShapeDtypeStruct((2 * N_PAD, H), jnp.float32),
               jax.ShapeDtypeStruct((N_PAD, D), jnp.bfloat16),
               jax.ShapeDtypeStruct((N_PAD, D), jnp.bfloat16)],
)


def _score_body(xl_hbm, xr_hbm, src_hbm, dst_hbm, att_hbm,
                eexp_hbm, den_hbm,
                src_v, dst_v, att_v,
                ll0, rl0, ll1, rl1,
                eexp_v, den_v, stage,
                g0, g1, g2, g3):
    c = lax.axis_index("c")
    s = lax.axis_index("s")
    wid = s * NC + c
    base = pl.multiple_of(wid * EP1, K1)

    pltpu.sync_copy(src_hbm.at[pl.ds(base, EP1)], src_v)
    pltpu.sync_copy(dst_hbm.at[pl.ds(base, EP1)], dst_v)
    pltpu.sync_copy(att_hbm, att_v)

    def zero_den(i, _):
        den_v[pl.ds(i * L, L)] = jnp.zeros((L,), jnp.float32)
        return 0

    lax.fori_loop(0, N_PAD // L, zero_den, 0)

    att_regs = []
    for i in range(D // (2 * L)):
        ab = plsc.bitcast(att_v[pl.ds(i * L, L)], jnp.bfloat16)
        att_regs.append(plsc.unpack(ab, format=plsc.PackFormat.INTERLEAVED))
    idx16 = lax.iota(jnp.int32, L)
    sets = ((ll0, rl0, g0, g1), (ll1, rl1, g2, g3))

    def descs(b, st):
        bl, rl, m0, m1 = st
        eb = b * K1
        return (
            pltpu.make_async_copy(xl_hbm.at[src_v.at[pl.ds(eb, K1)]], bl, m0),
            pltpu.make_async_copy(xr_hbm.at[dst_v.at[pl.ds(eb, K1)]], rl, m1),
        )

    def fire1(b, st):
        for d in descs(b, st):
            d.start()

    def wait1(b, st):
        for d in descs(b, st):
            d.wait()

    def compute(b, st):
        bl_, rl_ = st[0], st[1]
        eb = b * K1

        def group_body(g, _):
            def edge_body(jj, packvec):
                row = g * L + jj
                acc = jnp.zeros((L,), jnp.float32)
                for cidx in range(D // (2 * L)):
                    sl = pl.ds(cidx * L, L)
                    zb = (plsc.bitcast(bl_[row, sl], jnp.bfloat16)
                          + plsc.bitcast(rl_[row, sl], jnp.bfloat16))
                    lb = jnp.maximum(zb, zb * NEG)
                    z0, z1 = plsc.unpack(
                        lb, format=plsc.PackFormat.INTERLEAVED)
                    a0, a1 = att_regs[cidx]
                    acc = acc + z0 * a0 + z1 * a1
                for sh in (1, 2, 4, 8):
                    perm = jnp.bitwise_xor(idx16, sh)
                    acc = acc + acc.at[perm].get(mode="promise_in_bounds")
                return jnp.where(idx16 == jj, acc, packvec)

            packvec = lax.fori_loop(0, L, edge_body,
                                    jnp.zeros((L,), jnp.float32))
            eexp = jnp.exp(packvec)
            sl = pl.ds(eb + g * L, L)
            eexp_v[sl] = eexp
            plsc.addupdate_scatter(den_v, [dst_v[sl]], eexp)
            return 0

        lax.fori_loop(0, K1 // L, group_body, 0)

    fire1(0, sets[0])

    def pair_body(m, _):
        for q in range(2):
            b = m * 2 + q

            @pl.when(b + 1 < NB1)
            def _():
                fire1(b + 1, sets[1 - q])

            wait1(b, sets[q])
            compute(b, sets[q])
        return 0

    lax.fori_loop(0, NB1 // 2, pair_body, 0)

    pltpu.sync_copy(eexp_v, eexp_hbm.at[pl.ds(base, EP1)])

    pltpu.sync_copy(den_v, stage.at[s])
    plsc.subcore_barrier()
    myslice = pl.multiple_of(s * (N_PAD // NS), L)
    dacc = den_v.at[pl.ds(0, N_PAD // NS)]
    dtmp = den_v.at[pl.ds(N_PAD // NS, N_PAD // NS)]
    pltpu.sync_copy(stage.at[0, pl.ds(myslice, N_PAD // NS)], dacc)
    for t in range(1, NS):
        pltpu.sync_copy(stage.at[t, pl.ds(myslice, N_PAD // NS)], dtmp)
        for i in range(N_PAD // NS // L):
            sl = pl.ds(i * L, L)
            dacc[sl] = dacc[sl] + dtmp[sl]
    pltpu.sync_copy(dacc, den_hbm.at[c, pl.ds(myslice, N_PAD // NS)])


_score = pl.kernel(
    _score_body,
    out_type=[jax.ShapeDtypeStruct((E_PAD,), jnp.float32),
              jax.ShapeDtypeStruct((NC, N_PAD), jnp.float32)],
    mesh=plsc.VectorSubcoreMesh(core_axis_name="c", subcore_axis_name="s"),
    compiler_params=pltpu.CompilerParams(needs_layout_passes=False),
    scratch_types=(
        [pltpu.VMEM((EP1,), jnp.int32)] * 2
        + [pltpu.VMEM((D // 2,), jnp.float32)]
        + [pltpu.VMEM((K1, H), jnp.float32)] * 4
        + [pltpu.VMEM((EP1,), jnp.float32),
           pltpu.VMEM((N_PAD,), jnp.float32),
           pltpu.VMEM_SHARED((NS, N_PAD), jnp.float32)]
        + [pltpu.SemaphoreType.DMA] * 4
    ),
)


def _agg_body(xl_hbm, src3_hbm, dst3_hbm, eexp3_hbm, den_hbm, bias_hbm,
              out_hbm,
              a0, a1, a2, a3, d0, d1, d2, d3,
              e0, e1, e2, e3, r0, r1, r2, r3,
              den0_v, den1_v, bias_v, out_acc,
              sg0, sg1, sg2, sg3, ss0, ss1, ss2, ss3,
              se0, se1, se2, se3, sa0, sa1, sa2, sa3):
    c = lax.axis_index("c")
    s = lax.axis_index("s")

    coff = jnp.full((L,), c * N_PAD, jnp.int32)

    rows = (r0, r1, r2, r3)
    srcb = (a0, a1, a2, a3)
    dstb = (d0, d1, d2, d3)
    ebufs = (e0, e1, e2, e3)
    gsems = (sg0, sg1, sg2, sg3)
    ssems = (ss0, ss1, ss2, ss3)
    esems = (se0, se1, se2, se3)
    asems = (sa0, sa1, sa2, sa3)

    def zrow(j, _):
        for v in range(H // L):
            r0[j, pl.ds(v * L, L)] = jnp.zeros((L,), jnp.float32)
        return 0

    lax.fori_loop(0, DCH, zrow, 0)

    def zcopy(i, _):
        pltpu.sync_copy(r0.at[pl.ds(0, DCH)],
                        out_acc.at[pl.ds(s * DR + i * DCH, DCH)])
        return 0

    lax.fori_loop(0, DR // DCH, zcopy, 0)
    plsc.subcore_barrier()

    def gat_desc(q):
        return pltpu.make_async_copy(xl_hbm.at[srcb[q]], rows[q], gsems[q])

    def exp_desc(b, q):
        return pltpu.make_async_copy(eexp3_hbm.at[s, b], ebufs[q], esems[q])

    def src_desc(b, q):
        return pltpu.make_async_copy(src3_hbm.at[s, b], srcb[q], asems[q])

    def dst_desc(b, q):
        return pltpu.make_async_copy(dst3_hbm.at[s, b], dstb[q], esems[q])

    def sca_desc(q):
        return pltpu.make_async_copy(rows[q], out_acc.at[dstb[q]], ssems[q])

    def shift_fire_gather(q):
        for v in range(K2 // L):
            sl = pl.ds(v * L, L)
            srcb[q][sl] = srcb[q][sl] + coff
        gat_desc(q).start()

    def scale(q):
        rbuf, ebuf = rows[q], ebufs[q]
        for g in range(K2 // L):
            ev16 = ebuf[pl.ds(g * L, L)]

            def srow(jj, _):
                ev = ev16.at[jnp.full((L,), jj, jnp.int32)].get(
                    mode="promise_in_bounds")
                row = g * L + jj
                for v in range(H // L):
                    sl = pl.ds(v * L, L)
                    rbuf[row, sl] = rbuf[row, sl] * ev
                return 0

            lax.fori_loop(0, L, srow, 0)

    for b0 in range(4):
        src_desc(b0, b0).start()
    for b0 in range(2):
        exp_desc(b0, b0).start()
        dst_desc(b0, b0).start()
        src_desc(b0, b0).wait()
        shift_fire_gather(b0)

    def macro_body(m, _):
        for q in range(4):
            b = m * 4 + q
            q2 = (q + 2) % 4

            @pl.when(b >= 2)
            def _():
                sca_desc(q2).wait()

            @pl.when(b + 2 < NB2)
            def _():
                src_desc(b + 2, q2).wait()
                exp_desc(b + 2, q2).start()
                dst_desc(b + 2, q2).start()
                shift_fire_gather(q2)

            gat_desc(q).wait()
            exp_desc(b, q).wait()
            dst_desc(b, q).wait()

            @pl.when(b + 4 < NB2)
            def _():
                src_desc(b + 4, q).start()

            scale(q)
            pltpu.async_copy(rows[q], out_acc.at[dstb[q]], ssems[q],
                             add=True)
        return 0

    lax.fori_loop(0, NB2 // 4, macro_body, 0)
    sca_desc((NB2 - 2) % 4).wait()
    sca_desc((NB2 - 1) % 4).wait()
    plsc.subcore_barrier()

    myrow = pl.multiple_of(s * DR, L)
    pltpu.sync_copy(bias_hbm.at[c], bias_v)

    def drain_body(i, _):
        r0c = pl.multiple_of(myrow + i * DCH, L)
        pltpu.sync_copy(out_acc.at[pl.ds(r0c, DCH)], r0.at[pl.ds(0, DCH)])
        pltpu.sync_copy(den_hbm.at[0, pl.ds(r0c, DCH)],
                        den0_v.at[pl.ds(0, DCH)])
        pltpu.sync_copy(den_hbm.at[1, pl.ds(r0c, DCH)],
                        den1_v.at[pl.ds(0, DCH)])
        for g in range(DCH // L):
            sl = pl.ds(g * L, L)
            den0_v[sl] = den0_v[sl] + den1_v[sl]

        def dgroup(g, _):
            dv16 = den0_v[pl.ds(g * L, L)]

            def inner(jj, _):
                dv = dv16.at[jnp.full((L,), jj, jnp.int32)].get(
                    mode="promise_in_bounds")
                row = g * L + jj
                for v in range(H // L):
                    sl = pl.ds(v * L, L)
                    r0[row, sl] = r0[row, sl] / dv + bias_v[sl]
                return 0

            lax.fori_loop(0, L, inner, 0)
            return 0

        lax.fori_loop(0, DCH // L, dgroup, 0)
        pltpu.sync_copy(r0.at[pl.ds(0, DCH)],
                        out_hbm.at[pl.ds(c * N_PAD + r0c, DCH)])
        return 0

    lax.fori_loop(0, DR // DCH, drain_body, 0)


_agg = pl.kernel(
    _agg_body,
    out_type=jax.ShapeDtypeStruct((2 * N_PAD, H), jnp.float32),
    mesh=plsc.VectorSubcoreMesh(core_axis_name="c", subcore_axis_name="s"),
    compiler_params=pltpu.CompilerParams(needs_layout_passes=False),
    scratch_types=(
        [pltpu.VMEM((K2,), jnp.int32)] * 8
        + [pltpu.VMEM((K2,), jnp.float32)] * 4
        + [pltpu.VMEM((K2, H), jnp.float32)] * 4
        + [pltpu.VMEM((K2,), jnp.float32)] * 2
        + [pltpu.VMEM((H,), jnp.float32),
           pltpu.VMEM_SHARED((N_PAD, H), jnp.float32)]
        + [pltpu.SemaphoreType.DMA] * 16
    ),
)


def kernel(x, edge_index, W_l, b_l, W_r, b_r, att, bias):
    loops = jnp.arange(N, dtype=edge_index.dtype)
    src = jnp.concatenate(
        [edge_index[0], loops,
         jnp.zeros((E_PAD - E_TOT,), edge_index.dtype)])
    dst = jnp.concatenate(
        [edge_index[1], loops,
         jnp.full((E_PAD - E_TOT,), N, edge_index.dtype)])
    src = src.astype(jnp.int32)
    dst = dst.astype(jnp.int32)
    src3 = src.reshape(NS, NB2, K2)
    dst3 = dst.reshape(NS, NB2, K2)

    x_pad = jnp.pad(x, ((0, N_PAD - N), (0, 0)))
    xl_parts, xlb, xrb = _proj(x_pad, W_l, b_l.reshape(1, D),
                               W_r, b_r.reshape(1, D))

    xlb32 = lax.bitcast_convert_type(
        xlb.reshape(N_PAD, D // 2, 2), jnp.float32)
    xrb32 = lax.bitcast_convert_type(
        xrb.reshape(N_PAD, D // 2, 2), jnp.float32)
    att32 = lax.bitcast_convert_type(
        att.astype(jnp.bfloat16).reshape(D // 2, 2), jnp.float32)
    eexp, den_parts = _score(xlb32, xrb32, src, dst, att32)
    out_parts = _agg(xl_parts, src3, dst3, eexp.reshape(NS, NB2, K2),
                     den_parts, bias.reshape(NC, H))
    return jnp.concatenate(
        [out_parts[:N], out_parts[N_PAD:N_PAD + N]], axis=1)


# speedup vs baseline: 1.1746x; 1.1746x over previous
import jax
import jax.numpy as jnp
from jax import lax
from jax.experimental import pallas as pl
from jax.experimental.pallas import tpu as pltpu
from jax.experimental.pallas import tpu_sc as plsc

N = 10000
E = 160000
D = 256
H = 128
NEG = 0.2

L = 16
NC = 2
NS = 16
NW = NC * NS

N_PAD = 10240
E_TOT = E + N
E_PAD = 172032
K1 = 64
EP1 = E_PAD // NW
NB1 = EP1 // K1
K2 = 48
EP2 = E_PAD // NS
NB2 = EP2 // K2
R = 1024
DR = N_PAD // NS
DCH = 32


def _pack_bf16_pairs(v):
    u = lax.bitcast_convert_type(v.astype(jnp.bfloat16), jnp.uint16)
    lo = u[:, :H].astype(jnp.uint32)
    hi = u[:, H:].astype(jnp.uint32)
    return lax.bitcast_convert_type(lo | (hi << 16), jnp.float32)


def _proj_body(x_ref, wl_ref, bl_ref, wr_ref, br_ref, olb_ref, orb_ref):
    xb = x_ref[...]
    xl = jnp.dot(xb, wl_ref[...],
                 preferred_element_type=jnp.float32) + bl_ref[...]
    xr = jnp.dot(xb, wr_ref[...],
                 preferred_element_type=jnp.float32) + br_ref[...]
    olb_ref[...] = _pack_bf16_pairs(xl)
    orb_ref[...] = _pack_bf16_pairs(xr)


_proj = pl.pallas_call(
    _proj_body,
    grid=(N_PAD // R,),
    in_specs=[
        pl.BlockSpec((R, D), lambda i: (i, 0)),
        pl.BlockSpec((D, D), lambda i: (0, 0)),
        pl.BlockSpec((1, D), lambda i: (0, 0)),
        pl.BlockSpec((D, D), lambda i: (0, 0)),
        pl.BlockSpec((1, D), lambda i: (0, 0)),
    ],
    out_specs=[
        pl.BlockSpec((R, H), lambda i: (i, 0)),
        pl.BlockSpec((R, H), lambda i: (i, 0)),
    ],
    out_shape=[jax.ShapeDtypeStruct((N_PAD, H), jnp.float32)] * 2,
)


def _score_body(xl_hbm, xr_hbm, src_hbm, dst_hbm, att_hbm,
                eexp_hbm, den_hbm,
                src_v, dst_v, att_v,
                ll0, rl0, ll1, rl1,
                eexp_v, den_v, stage,
                g0, g1, g2, g3):
    c = lax.axis_index("c")
    s = lax.axis_index("s")
    wid = s * NC + c
    base = pl.multiple_of(wid * EP1, K1)

    pltpu.sync_copy(src_hbm.at[pl.ds(base, EP1)], src_v)
    pltpu.sync_copy(dst_hbm.at[pl.ds(base, EP1)], dst_v)
    pltpu.sync_copy(att_hbm, att_v)

    def zero_den(i, _):
        den_v[pl.ds(i * L, L)] = jnp.zeros((L,), jnp.float32)
        return 0

    lax.fori_loop(0, N_PAD // L, zero_den, 0)

    att_regs = []
    for i in range(D // (2 * L)):
        ab = plsc.bitcast(att_v[pl.ds(i * L, L)], jnp.bfloat16)
        att_regs.append(plsc.unpack(ab, format=plsc.PackFormat.INTERLEAVED))
    idx16 = lax.iota(jnp.int32, L)
    sets = ((ll0, rl0, g0, g1), (ll1, rl1, g2, g3))

    def descs(b, st):
        bl, rl, m0, m1 = st
        eb = b * K1
        return (
            pltpu.make_async_copy(xl_hbm.at[src_v.at[pl.ds(eb, K1)]], bl, m0),
            pltpu.make_async_copy(xr_hbm.at[dst_v.at[pl.ds(eb, K1)]], rl, m1),
        )

    def fire1(b, st):
        for d in descs(b, st):
            d.start()

    def wait1(b, st):
        for d in descs(b, st):
            d.wait()

    def compute(b, st):
        bl_, rl_ = st[0], st[1]
        eb = b * K1

        def group_body(g, _):
            def edge_body(jj, packvec):
                row = g * L + jj
                acc = jnp.zeros((L,), jnp.float32)
                for cidx in range(D // (2 * L)):
                    sl = pl.ds(cidx * L, L)
                    zb = (plsc.bitcast(bl_[row, sl], jnp.bfloat16)
                          + plsc.bitcast(rl_[row, sl], jnp.bfloat16))
                    lb = jnp.maximum(zb, zb * NEG)
                    z0, z1 = plsc.unpack(
                        lb, format=plsc.PackFormat.INTERLEAVED)
                    a0, a1 = att_regs[cidx]
                    acc = acc + z0 * a0 + z1 * a1
                for sh in (1, 2, 4, 8):
                    perm = jnp.bitwise_xor(idx16, sh)
                    acc = acc + acc.at[perm].get(mode="promise_in_bounds")
                return jnp.where(idx16 == jj, acc, packvec)

            packvec = lax.fori_loop(0, L, edge_body,
                                    jnp.zeros((L,), jnp.float32))
            eexp = jnp.exp(packvec)
            sl = pl.ds(eb + g * L, L)
            eexp_v[sl] = eexp
            plsc.addupdate_scatter(den_v, [dst_v[sl]], eexp)
            return 0

        lax.fori_loop(0, K1 // L, group_body, 0)

    fire1(0, sets[0])

    def pair_body(m, _):
        for q in range(2):
            b = m * 2 + q

            @pl.when(b + 1 < NB1)
            def _():
                fire1(b + 1, sets[1 - q])

            wait1(b, sets[q])
            compute(b, sets[q])
        return 0

    lax.fori_loop(0, NB1 // 2, pair_body, 0)

    pltpu.sync_copy(eexp_v, eexp_hbm.at[pl.ds(base, EP1)])

    pltpu.sync_copy(den_v, stage.at[s])
    plsc.subcore_barrier()
    myslice = pl.multiple_of(s * (N_PAD // NS), L)
    dacc = den_v.at[pl.ds(0, N_PAD // NS)]
    dtmp = den_v.at[pl.ds(N_PAD // NS, N_PAD // NS)]
    pltpu.sync_copy(stage.at[0, pl.ds(myslice, N_PAD // NS)], dacc)
    for t in range(1, NS):
        pltpu.sync_copy(stage.at[t, pl.ds(myslice, N_PAD // NS)], dtmp)
        for i in range(N_PAD // NS // L):
            sl = pl.ds(i * L, L)
            dacc[sl] = dacc[sl] + dtmp[sl]
    pltpu.sync_copy(dacc, den_hbm.at[c, pl.ds(myslice, N_PAD // NS)])


_score = pl.kernel(
    _score_body,
    out_type=[jax.ShapeDtypeStruct((E_PAD,), jnp.float32),
              jax.ShapeDtypeStruct((NC, N_PAD), jnp.float32)],
    mesh=plsc.VectorSubcoreMesh(core_axis_name="c", subcore_axis_name="s"),
    compiler_params=pltpu.CompilerParams(needs_layout_passes=False),
    scratch_types=(
        [pltpu.VMEM((EP1,), jnp.int32)] * 2
        + [pltpu.VMEM((D // 2,), jnp.float32)]
        + [pltpu.VMEM((K1, H), jnp.float32)] * 4
        + [pltpu.VMEM((EP1,), jnp.float32),
           pltpu.VMEM((N_PAD,), jnp.float32),
           pltpu.VMEM_SHARED((NS, N_PAD), jnp.float32)]
        + [pltpu.SemaphoreType.DMA] * 4
    ),
)


def _agg_body(xl_hbm, src3_hbm, dst3_hbm, eexp3_hbm, den_hbm, bias_hbm,
              out_hbm,
              a0, a1, a2, a3, d0, d1, d2, d3,
              e0, e1, e2, e3, r0, r1, r2, r3,
              den0_v, den1_v, bias_v, out_acc,
              sg0, sg1, sg2, sg3, ss0, ss1, ss2, ss3,
              se0, se1, se2, se3, sa0, sa1, sa2, sa3):
    c = lax.axis_index("c")
    s = lax.axis_index("s")

    rows = (r0, r1, r2, r3)
    srcb = (a0, a1, a2, a3)
    dstb = (d0, d1, d2, d3)
    ebufs = (e0, e1, e2, e3)
    gsems = (sg0, sg1, sg2, sg3)
    ssems = (ss0, ss1, ss2, ss3)
    esems = (se0, se1, se2, se3)
    asems = (sa0, sa1, sa2, sa3)

    def zrow(j, _):
        for v in range(H // L):
            r0[j, pl.ds(v * L, L)] = jnp.zeros((L,), jnp.float32)
        return 0

    lax.fori_loop(0, DCH, zrow, 0)

    def zcopy(i, _):
        pltpu.sync_copy(r0.at[pl.ds(0, DCH)],
                        out_acc.at[pl.ds(s * DR + i * DCH, DCH)])
        return 0

    lax.fori_loop(0, DR // DCH, zcopy, 0)
    plsc.subcore_barrier()

    def gat_desc(q):
        return pltpu.make_async_copy(xl_hbm.at[srcb[q]], rows[q], gsems[q])

    def exp_desc(b, q):
        return pltpu.make_async_copy(eexp3_hbm.at[s, b], ebufs[q], esems[q])

    def src_desc(b, q):
        return pltpu.make_async_copy(src3_hbm.at[s, b], srcb[q], asems[q])

    def dst_desc(b, q):
        return pltpu.make_async_copy(dst3_hbm.at[s, b], dstb[q], esems[q])

    def sca_desc(q):
        return pltpu.make_async_copy(rows[q], out_acc.at[dstb[q]], ssems[q])

    def scale(q):
        rbuf, ebuf = rows[q], ebufs[q]
        czero = c == 0
        for g in range(K2 // L):
            ev16 = ebuf[pl.ds(g * L, L)]

            def srow(jj, _):
                ev = ev16.at[jnp.full((L,), jj, jnp.int32)].get(
                    mode="promise_in_bounds")
                row = g * L + jj
                for v in range(H // L):
                    sl = pl.ds(v * L, L)
                    zb = plsc.bitcast(rbuf[row, sl], jnp.bfloat16)
                    z0, z1 = plsc.unpack(
                        zb, format=plsc.PackFormat.INTERLEAVED)
                    zc = jnp.where(czero, z0, z1)
                    rbuf[row, sl] = zc * ev
                return 0

            lax.fori_loop(0, L, srow, 0)

    for b0 in range(4):
        src_desc(b0, b0).start()
    for b0 in range(2):
        exp_desc(b0, b0).start()
        dst_desc(b0, b0).start()
        src_desc(b0, b0).wait()
        gat_desc(b0).start()

    def macro_body(m, _):
        for q in range(4):
            b = m * 4 + q
            q2 = (q + 2) % 4

            @pl.when(b >= 2)
            def _():
                sca_desc(q2).wait()

            @pl.when(b + 2 < NB2)
            def _():
                src_desc(b + 2, q2).wait()
                exp_desc(b + 2, q2).start()
                dst_desc(b + 2, q2).start()
                gat_desc(q2).start()

            gat_desc(q).wait()
            exp_desc(b, q).wait()
            dst_desc(b, q).wait()

            @pl.when(b + 4 < NB2)
            def _():
                src_desc(b + 4, q).start()

            scale(q)
            pltpu.async_copy(rows[q], out_acc.at[dstb[q]], ssems[q],
                             add=True)
        return 0

    lax.fori_loop(0, NB2 // 4, macro_body, 0)
    sca_desc((NB2 - 2) % 4).wait()
    sca_desc((NB2 - 1) % 4).wait()
    plsc.subcore_barrier()

    myrow = pl.multiple_of(s * DR, L)
    pltpu.sync_copy(bias_hbm.at[c], bias_v)

    def drain_body(i, _):
        r0c = pl.multiple_of(myrow + i * DCH, L)
        pltpu.sync_copy(out_acc.at[pl.ds(r0c, DCH)], r0.at[pl.ds(0, DCH)])
        pltpu.sync_copy(den_hbm.at[0, pl.ds(r0c, DCH)],
                        den0_v.at[pl.ds(0, DCH)])
        pltpu.sync_copy(den_hbm.at[1, pl.ds(r0c, DCH)],
                        den1_v.at[pl.ds(0, DCH)])
        for g in range(DCH // L):
            sl = pl.ds(g * L, L)
            den0_v[sl] = den0_v[sl] + den1_v[sl]

        def dgroup(g, _):
            dv16 = den0_v[pl.ds(g * L, L)]

            def inner(jj, _):
                dv = dv16.at[jnp.full((L,), jj, jnp.int32)].get(
                    mode="promise_in_bounds")
                row = g * L + jj
                for v in range(H // L):
                    sl = pl.ds(v * L, L)
                    r0[row, sl] = r0[row, sl] / dv + bias_v[sl]
                return 0

            lax.fori_loop(0, L, inner, 0)
            return 0

        lax.fori_loop(0, DCH // L, dgroup, 0)
        pltpu.sync_copy(r0.at[pl.ds(0, DCH)],
                        out_hbm.at[pl.ds(c * N_PAD + r0c, DCH)])
        return 0

    lax.fori_loop(0, DR // DCH, drain_body, 0)


_agg = pl.kernel(
    _agg_body,
    out_type=jax.ShapeDtypeStruct((2 * N_PAD, H), jnp.float32),
    mesh=plsc.VectorSubcoreMesh(core_axis_name="c", subcore_axis_name="s"),
    compiler_params=pltpu.CompilerParams(needs_layout_passes=False),
    scratch_types=(
        [pltpu.VMEM((K2,), jnp.int32)] * 8
        + [pltpu.VMEM((K2,), jnp.float32)] * 4
        + [pltpu.VMEM((K2, H), jnp.float32)] * 4
        + [pltpu.VMEM((K2,), jnp.float32)] * 2
        + [pltpu.VMEM((H,), jnp.float32),
           pltpu.VMEM_SHARED((N_PAD, H), jnp.float32)]
        + [pltpu.SemaphoreType.DMA] * 16
    ),
)


def kernel(x, edge_index, W_l, b_l, W_r, b_r, att, bias):
    loops = jnp.arange(N, dtype=edge_index.dtype)
    src = jnp.concatenate(
        [edge_index[0], loops,
         jnp.zeros((E_PAD - E_TOT,), edge_index.dtype)])
    dst = jnp.concatenate(
        [edge_index[1], loops,
         jnp.full((E_PAD - E_TOT,), N, edge_index.dtype)])
    src = src.astype(jnp.int32)
    dst = dst.astype(jnp.int32)
    src3 = src.reshape(NS, NB2, K2)
    dst3 = dst.reshape(NS, NB2, K2)

    x_pad = jnp.pad(x, ((0, N_PAD - N), (0, 0)))
    xlb, xrb = _proj(x_pad, W_l, b_l.reshape(1, D),
                     W_r, b_r.reshape(1, D))

    au = lax.bitcast_convert_type(att.astype(jnp.bfloat16), jnp.uint16)
    att32 = lax.bitcast_convert_type(
        au[:H].astype(jnp.uint32) | (au[H:].astype(jnp.uint32) << 16),
        jnp.float32)
    eexp, den_parts = _score(xlb, xrb, src, dst, att32)
    out_parts = _agg(xlb, src3, dst3, eexp.reshape(NS, NB2, K2),
                     den_parts, bias.reshape(NC, H))
    return jnp.concatenate(
        [out_parts[:N], out_parts[N_PAD:N_PAD + N]], axis=1)


# speedup vs baseline: 1.1952x; 1.0175x over previous
import jax
import jax.numpy as jnp
from jax import lax
from jax.experimental import pallas as pl
from jax.experimental.pallas import tpu as pltpu
from jax.experimental.pallas import tpu_sc as plsc

N = 10000
E = 160000
D = 256
H = 128
NEG = 0.2

L = 16
NC = 2
NS = 16
NW = NC * NS

N_PAD = 10240
E_TOT = E + N
E_PAD = 172032
K1 = 64
EP1 = E_PAD // NW
NB1 = EP1 // K1
K2 = 48
EP2 = E_PAD // NS
NB2 = EP2 // K2
R = 1024
DR = N_PAD // NS
DCH = 32


def _pack_bf16_pairs(v):
    u = lax.bitcast_convert_type(v.astype(jnp.bfloat16), jnp.uint16)
    lo = u[:, :H].astype(jnp.uint32)
    hi = u[:, H:].astype(jnp.uint32)
    return lax.bitcast_convert_type(lo | (hi << 16), jnp.float32)


def _proj_body(x_ref, wl_ref, bl_ref, wr_ref, br_ref, olb_ref, orb_ref):
    xb = x_ref[...]
    xl = jnp.dot(xb, wl_ref[...],
                 preferred_element_type=jnp.float32) + bl_ref[...]
    xr = jnp.dot(xb, wr_ref[...],
                 preferred_element_type=jnp.float32) + br_ref[...]
    olb_ref[...] = _pack_bf16_pairs(xl)
    orb_ref[...] = _pack_bf16_pairs(xr)


_proj = pl.pallas_call(
    _proj_body,
    grid=(N_PAD // R,),
    in_specs=[
        pl.BlockSpec((R, D), lambda i: (i, 0)),
        pl.BlockSpec((D, D), lambda i: (0, 0)),
        pl.BlockSpec((1, D), lambda i: (0, 0)),
        pl.BlockSpec((D, D), lambda i: (0, 0)),
        pl.BlockSpec((1, D), lambda i: (0, 0)),
    ],
    out_specs=[
        pl.BlockSpec((R, H), lambda i: (i, 0)),
        pl.BlockSpec((R, H), lambda i: (i, 0)),
    ],
    out_shape=[jax.ShapeDtypeStruct((N_PAD, H), jnp.float32)] * 2,
)


def _score_body(xl_hbm, xr_hbm, src_hbm, dst_hbm, att_hbm,
                eexp_hbm, den_hbm,
                src_v, dst_v, att_v,
                ll0, rl0, ll1, rl1,
                eexp_v, den_v, stage,
                g0, g1, g2, g3):
    c = lax.axis_index("c")
    s = lax.axis_index("s")
    wid = s * NC + c
    base = pl.multiple_of(wid * EP1, K1)

    pltpu.sync_copy(src_hbm.at[pl.ds(base, EP1)], src_v)
    pltpu.sync_copy(dst_hbm.at[pl.ds(base, EP1)], dst_v)
    pltpu.sync_copy(att_hbm, att_v)

    def zero_den(i, _):
        den_v[pl.ds(i * L, L)] = jnp.zeros((L,), jnp.float32)
        return 0

    lax.fori_loop(0, N_PAD // L, zero_den, 0)

    att_regs = []
    for i in range(D // (2 * L)):
        ab = plsc.bitcast(att_v[pl.ds(i * L, L)], jnp.bfloat16)
        att_regs.append(plsc.unpack(ab, format=plsc.PackFormat.INTERLEAVED))
    idx16 = lax.iota(jnp.int32, L)
    sets = ((ll0, rl0, g0, g1), (ll1, rl1, g2, g3))

    def descs(b, st):
        bl, rl, m0, m1 = st
        eb = b * K1
        return (
            pltpu.make_async_copy(xl_hbm.at[src_v.at[pl.ds(eb, K1)]], bl, m0),
            pltpu.make_async_copy(xr_hbm.at[dst_v.at[pl.ds(eb, K1)]], rl, m1),
        )

    def fire1(b, st):
        for d in descs(b, st):
            d.start()

    def wait1(b, st):
        for d in descs(b, st):
            d.wait()

    def compute(b, st):
        bl_, rl_ = st[0], st[1]
        eb = b * K1

        def group_body(g, _):
            def edge_body(jj, packvec):
                row = g * L + jj
                acc = jnp.zeros((L,), jnp.float32)
                for cidx in range(D // (2 * L)):
                    sl = pl.ds(cidx * L, L)
                    zb = (plsc.bitcast(bl_[row, sl], jnp.bfloat16)
                          + plsc.bitcast(rl_[row, sl], jnp.bfloat16))
                    lb = jnp.maximum(zb, zb * NEG)
                    z0, z1 = plsc.unpack(
                        lb, format=plsc.PackFormat.INTERLEAVED)
                    a0, a1 = att_regs[cidx]
                    acc = acc + z0 * a0 + z1 * a1
                for sh in (1, 2, 4, 8):
                    perm = jnp.bitwise_xor(idx16, sh)
                    acc = acc + acc.at[perm].get(mode="promise_in_bounds")
                return jnp.where(idx16 == jj, acc, packvec)

            packvec = lax.fori_loop(0, L, edge_body,
                                    jnp.zeros((L,), jnp.float32))
            eexp = jnp.exp(packvec)
            sl = pl.ds(eb + g * L, L)
            eexp_v[sl] = eexp
            plsc.addupdate_scatter(den_v, [dst_v[sl]], eexp)
            return 0

        lax.fori_loop(0, K1 // L, group_body, 0)

    fire1(0, sets[0])

    def pair_body(m, _):
        for q in range(2):
            b = m * 2 + q

            @pl.when(b + 1 < NB1)
            def _():
                fire1(b + 1, sets[1 - q])

            wait1(b, sets[q])
            compute(b, sets[q])
        return 0

    lax.fori_loop(0, NB1 // 2, pair_body, 0)

    pltpu.sync_copy(eexp_v, eexp_hbm.at[pl.ds(base, EP1)])

    pltpu.sync_copy(den_v, stage.at[s])
    plsc.subcore_barrier()
    myslice = pl.multiple_of(s * (N_PAD // NS), L)
    dacc = den_v.at[pl.ds(0, N_PAD // NS)]
    dtmp = den_v.at[pl.ds(N_PAD // NS, N_PAD // NS)]
    pltpu.sync_copy(stage.at[0, pl.ds(myslice, N_PAD // NS)], dacc)
    for t in range(1, NS):
        pltpu.sync_copy(stage.at[t, pl.ds(myslice, N_PAD // NS)], dtmp)
        for i in range(N_PAD // NS // L):
            sl = pl.ds(i * L, L)
            dacc[sl] = dacc[sl] + dtmp[sl]
    pltpu.sync_copy(dacc, den_hbm.at[c, pl.ds(myslice, N_PAD // NS)])


_score = pl.kernel(
    _score_body,
    out_type=[jax.ShapeDtypeStruct((E_PAD,), jnp.float32),
              jax.ShapeDtypeStruct((NC, N_PAD), jnp.float32)],
    mesh=plsc.VectorSubcoreMesh(core_axis_name="c", subcore_axis_name="s"),
    compiler_params=pltpu.CompilerParams(needs_layout_passes=False),
    scratch_types=(
        [pltpu.VMEM((EP1,), jnp.int32)] * 2
        + [pltpu.VMEM((D // 2,), jnp.float32)]
        + [pltpu.VMEM((K1, H), jnp.float32)] * 4
        + [pltpu.VMEM((EP1,), jnp.float32),
           pltpu.VMEM((N_PAD,), jnp.float32),
           pltpu.VMEM_SHARED((NS, N_PAD), jnp.float32)]
        + [pltpu.SemaphoreType.DMA] * 4
    ),
)


def _agg_body(xl_hbm, src3_hbm, dst3_hbm, eexp3_hbm, den_hbm, bias_hbm,
              out_hbm,
              a0, a1, a2, a3, d0, d1, d2, d3,
              e0, e1, e2, e3, r0, r1, r2, r3,
              den0_v, den1_v, bias_v, out_acc,
              sg0, sg1, sg2, sg3, ss0, ss1, ss2, ss3,
              se0, se1, se2, se3, sa0, sa1, sa2, sa3):
    c = lax.axis_index("c")
    s = lax.axis_index("s")

    rows = (r0, r1, r2, r3)
    srcb = (a0, a1, a2, a3)
    dstb = (d0, d1, d2, d3)
    ebufs = (e0, e1, e2, e3)
    gsems = (sg0, sg1, sg2, sg3)
    ssems = (ss0, ss1, ss2, ss3)
    esems = (se0, se1, se2, se3)
    asems = (sa0, sa1, sa2, sa3)

    def zrow(j, _):
        for v in range(H // L):
            r0[j, pl.ds(v * L, L)] = jnp.zeros((L,), jnp.float32)
        return 0

    lax.fori_loop(0, DCH, zrow, 0)

    def zcopy(i, _):
        pltpu.sync_copy(r0.at[pl.ds(0, DCH)],
                        out_acc.at[pl.ds(s * DR + i * DCH, DCH)])
        return 0

    lax.fori_loop(0, DR // DCH, zcopy, 0)
    plsc.subcore_barrier()

    def gat_desc(q):
        return pltpu.make_async_copy(xl_hbm.at[srcb[q]], rows[q], gsems[q])

    def exp_desc(b, q):
        return pltpu.make_async_copy(eexp3_hbm.at[s, b], ebufs[q], esems[q])

    def src_desc(b, q):
        return pltpu.make_async_copy(src3_hbm.at[s, b], srcb[q], asems[q])

    def dst_desc(b, q):
        return pltpu.make_async_copy(dst3_hbm.at[s, b], dstb[q], esems[q])

    def sca_desc(q):
        return pltpu.make_async_copy(rows[q], out_acc.at[dstb[q]], ssems[q])

    shamt = jnp.full((L,), 16 * (1 - c), jnp.int32)
    himask = jnp.full((L,), jnp.int32(-65536))

    def scale(q):
        rbuf, ebuf = rows[q], ebufs[q]
        for g in range(K2 // L):
            ev16 = ebuf[pl.ds(g * L, L)]

            def srow(jj, _):
                ev = ev16.at[jnp.full((L,), jj, jnp.int32)].get(
                    mode="promise_in_bounds")
                row = g * L + jj
                for v in range(H // L):
                    sl = pl.ds(v * L, L)
                    u = plsc.bitcast(rbuf[row, sl], jnp.int32)
                    zc = plsc.bitcast(
                        jnp.bitwise_and(jnp.left_shift(u, shamt), himask),
                        jnp.float32)
                    rbuf[row, sl] = zc * ev
                return 0

            lax.fori_loop(0, L, srow, 0)

    for b0 in range(4):
        src_desc(b0, b0).start()
    for b0 in range(2):
        exp_desc(b0, b0).start()
        dst_desc(b0, b0).start()
        src_desc(b0, b0).wait()
        gat_desc(b0).start()

    def macro_body(m, _):
        for q in range(4):
            b = m * 4 + q
            q2 = (q + 2) % 4

            @pl.when(b >= 2)
            def _():
                sca_desc(q2).wait()

            @pl.when(b + 2 < NB2)
            def _():
                src_desc(b + 2, q2).wait()
                exp_desc(b + 2, q2).start()
                dst_desc(b + 2, q2).start()
                gat_desc(q2).start()

            gat_desc(q).wait()
            exp_desc(b, q).wait()
            dst_desc(b, q).wait()

            @pl.when(b + 4 < NB2)
            def _():
                src_desc(b + 4, q).start()

            scale(q)
            pltpu.async_copy(rows[q], out_acc.at[dstb[q]], ssems[q],
                             add=True)
        return 0

    lax.fori_loop(0, NB2 // 4, macro_body, 0)
    sca_desc((NB2 - 2) % 4).wait()
    sca_desc((NB2 - 1) % 4).wait()
    plsc.subcore_barrier()

    myrow = pl.multiple_of(s * DR, L)
    pltpu.sync_copy(bias_hbm.at[c], bias_v)

    def drain_body(i, _):
        r0c = pl.multiple_of(myrow + i * DCH, L)
        pltpu.sync_copy(out_acc.at[pl.ds(r0c, DCH)], r0.at[pl.ds(0, DCH)])
        pltpu.sync_copy(den_hbm.at[0, pl.ds(r0c, DCH)],
                        den0_v.at[pl.ds(0, DCH)])
        pltpu.sync_copy(den_hbm.at[1, pl.ds(r0c, DCH)],
                        den1_v.at[pl.ds(0, DCH)])
        for g in range(DCH // L):
            sl = pl.ds(g * L, L)
            den0_v[sl] = den0_v[sl] + den1_v[sl]

        def dgroup(g, _):
            dv16 = den0_v[pl.ds(g * L, L)]

            def inner(jj, _):
                dv = dv16.at[jnp.full((L,), jj, jnp.int32)].get(
                    mode="promise_in_bounds")
                row = g * L + jj
                for v in range(H // L):
                    sl = pl.ds(v * L, L)
                    r0[row, sl] = r0[row, sl] / dv + bias_v[sl]
                return 0

            lax.fori_loop(0, L, inner, 0)
            return 0

        lax.fori_loop(0, DCH // L, dgroup, 0)
        pltpu.sync_copy(r0.at[pl.ds(0, DCH)],
                        out_hbm.at[pl.ds(c * N_PAD + r0c, DCH)])
        return 0

    lax.fori_loop(0, DR // DCH, drain_body, 0)


_agg = pl.kernel(
    _agg_body,
    out_type=jax.ShapeDtypeStruct((2 * N_PAD, H), jnp.float32),
    mesh=plsc.VectorSubcoreMesh(core_axis_name="c", subcore_axis_name="s"),
    compiler_params=pltpu.CompilerParams(needs_layout_passes=False),
    scratch_types=(
        [pltpu.VMEM((K2,), jnp.int32)] * 8
        + [pltpu.VMEM((K2,), jnp.float32)] * 4
        + [pltpu.VMEM((K2, H), jnp.float32)] * 4
        + [pltpu.VMEM((K2,), jnp.float32)] * 2
        + [pltpu.VMEM((H,), jnp.float32),
           pltpu.VMEM_SHARED((N_PAD, H), jnp.float32)]
        + [pltpu.SemaphoreType.DMA] * 16
    ),
)


def kernel(x, edge_index, W_l, b_l, W_r, b_r, att, bias):
    loops = jnp.arange(N, dtype=edge_index.dtype)
    src = jnp.concatenate(
        [edge_index[0], loops,
         jnp.zeros((E_PAD - E_TOT,), edge_index.dtype)])
    dst = jnp.concatenate(
        [edge_index[1], loops,
         jnp.full((E_PAD - E_TOT,), N, edge_index.dtype)])
    src = src.astype(jnp.int32)
    dst = dst.astype(jnp.int32)
    src3 = src.reshape(NS, NB2, K2)
    dst3 = dst.reshape(NS, NB2, K2)

    x_pad = jnp.pad(x, ((0, N_PAD - N), (0, 0)))
    xlb, xrb = _proj(x_pad, W_l, b_l.reshape(1, D),
                     W_r, b_r.reshape(1, D))

    au = lax.bitcast_convert_type(att.astype(jnp.bfloat16), jnp.uint16)
    att32 = lax.bitcast_convert_type(
        au[:H].astype(jnp.uint32) | (au[H:].astype(jnp.uint32) << 16),
        jnp.float32)
    eexp, den_parts = _score(xlb, xrb, src, dst, att32)
    out_parts = _agg(xlb, src3, dst3, eexp.reshape(NS, NB2, K2),
                     den_parts, bias.reshape(NC, H))
    return jnp.concatenate(
        [out_parts[:N], out_parts[N_PAD:N_PAD + N]], axis=1)
